# baseline (speedup 1.0000x reference)
import jax
import jax.numpy as jnp
from jax.experimental import pallas as pl

_YOLO_LOSS_BIAS = 5.0
_ROWS = 2048


def _loss_kernel(pred_ref, lab_ref, bce_ref, ce_ref, cnt_ref):
    i = pl.program_id(0)

    x = pred_ref[...]
    lab = lab_ref[...]
    rows, width = x.shape

    obj_t = lab[:, 0:1].astype(jnp.float32)
    tgt = lab[:, 1:2]

    col = jax.lax.broadcasted_iota(jnp.int32, (rows, width), 1)
    is_cls = col >= 1

    neg = jnp.float32(-3.0e38)
    xm = jnp.where(is_cls, x, neg)
    m = jnp.max(xm, axis=1, keepdims=True)
    e = jnp.where(is_cls, jnp.exp(x - m), 0.0)
    s = jnp.sum(e, axis=1, keepdims=True)
    logz = m + jnp.log(s)

    onehot = col == (tgt + 1)
    tgt_logit = jnp.sum(jnp.where(onehot, x, 0.0), axis=1, keepdims=True)

    ce_row = (logz - tgt_logit) * obj_t

    obj_logit = x[:, 0:1]
    bce_row = (jnp.maximum(obj_logit, 0.0) - obj_logit * obj_t
               + jnp.log1p(jnp.exp(-jnp.abs(obj_logit))))

    bce_part = jnp.sum(bce_row).reshape(1, 1)
    ce_part = jnp.sum(ce_row).reshape(1, 1)
    cnt_part = jnp.sum(obj_t).reshape(1, 1)

    @pl.when(i == 0)
    def _init():
        zero = jnp.zeros((1, 1), jnp.float32)
        bce_ref[...] = zero
        ce_ref[...] = zero
        cnt_ref[...] = zero

    bce_ref[...] += bce_part
    ce_ref[...] += ce_part
    cnt_ref[...] += cnt_part


@jax.jit
def kernel(predictions, labels):
    n, width = predictions.shape
    rows = _ROWS
    grid = n // rows

    out_shape = [jax.ShapeDtypeStruct((1, 1), jnp.float32)] * 3
    bce_sum, ce_sum, cnt = pl.pallas_call(
        _loss_kernel,
        grid=(grid,),
        in_specs=[
            pl.BlockSpec((rows, width), lambda i: (i, 0)),
            pl.BlockSpec((rows, 2), lambda i: (i, 0)),
        ],
        out_specs=[
            pl.BlockSpec((1, 1), lambda i: (0, 0)),
            pl.BlockSpec((1, 1), lambda i: (0, 0)),
            pl.BlockSpec((1, 1), lambda i: (0, 0)),
        ],
        out_shape=out_shape,
    )(predictions, labels.astype(jnp.int32))

    bce = bce_sum[0, 0] / n
    ce = ce_sum[0, 0] / jnp.maximum(cnt[0, 0], 1.0)
    return _YOLO_LOSS_BIAS * bce + ce
